# baseline (speedup 1.0000x reference)
import functools

import jax
import jax.numpy as jnp
from jax import lax
from jax.experimental import pallas as pl
from jax.experimental.pallas import tpu as pltpu
from jax.experimental.pallas import tpu_sc as plsc

N = 10000
N0 = 10000
M = 32
D_IN = 128
D_OUT = 128
K = 15
POINT_INFLUENCE = 0.05

NW = 32
E = N * M
EW = E // NW
CH = 400
NCH = EW // CH

BQ = 200
EB = BQ * M
GRID = N // BQ
GQ = 8
NG = BQ // GQ
GE = GQ * M


@functools.partial(
    pl.kernel,
    out_type=[
        jax.ShapeDtypeStruct((E, D_IN), jnp.bfloat16),
        jax.ShapeDtypeStruct((E, 16), jnp.float32),
    ],
    mesh=plsc.VectorSubcoreMesh(core_axis_name="c", subcore_axis_name="s"),
    compiler_params=pltpu.CompilerParams(use_tc_tiling_on_sc=False),
    scratch_types=[
        pltpu.VMEM((2, CH), jnp.int32),
        pltpu.VMEM((CH, D_IN), jnp.bfloat16),
        pltpu.VMEM((CH, D_IN), jnp.bfloat16),
        pltpu.VMEM((CH, 16), jnp.float32),
        pltpu.VMEM((CH, 16), jnp.float32),
        pltpu.SemaphoreType.DMA,
        pltpu.SemaphoreType.DMA,
        pltpu.SemaphoreType.DMA,
        pltpu.SemaphoreType.DMA,
    ],
)
def _sc_gather(nb_hbm, x_hbm, sp16_hbm, g_out, p_out,
               idx_v, gbuf0, gbuf1, pbuf0, pbuf1, sg0, sg1, sp0, sp1):
    wid = lax.axis_index("s") * 2 + lax.axis_index("c")
    base = wid * EW
    gbufs = (gbuf0, gbuf1)
    pbufs = (pbuf0, pbuf1)
    sgs = (sg0, sg1)
    sps = (sp0, sp1)

    cps = [None, None]
    for i in range(NCH):
        b = i % 2
        off = pl.multiple_of(base + i * CH, 8)
        pltpu.sync_copy(nb_hbm.at[pl.ds(off, CH)], idx_v.at[b])
        cps[b] = (
            pltpu.async_copy(x_hbm.at[idx_v.at[b]], gbufs[b], sgs[b]),
            pltpu.async_copy(sp16_hbm.at[idx_v.at[b]], pbufs[b], sps[b]),
            off,
        )
        if i > 0:
            cpg, cpp, poff = cps[1 - b]
            cpg.wait()
            cpp.wait()
            pltpu.sync_copy(gbufs[1 - b], g_out.at[pl.ds(poff, CH)])
            pltpu.sync_copy(pbufs[1 - b], p_out.at[pl.ds(poff, CH)])
    b = (NCH - 1) % 2
    cpg, cpp, poff = cps[b]
    cpg.wait()
    cpp.wait()
    pltpu.sync_copy(gbufs[b], g_out.at[pl.ds(poff, CH)])
    pltpu.sync_copy(pbufs[b], p_out.at[pl.ds(poff, CH)])


def _tc_body(g_ref, p_ref, qr_ref, kpt_ref, ones_ref, kpsq_ref, rep_ref,
             mask_ref, w_ref, o_ref):
    d = p_ref[...] - qr_ref[...]
    cross = lax.dot_general(
        d, kpt_ref[...], (((1,), (0,)), ((), ())),
        precision=lax.Precision.HIGHEST,
        preferred_element_type=jnp.float32,
    )
    ddr = lax.dot_general(
        d * d, ones_ref[...], (((1,), (0,)), ((), ())),
        precision=lax.Precision.HIGHEST,
        preferred_element_type=jnp.float32,
    )
    sq = jnp.maximum(ddr - 2.0 * cross + kpsq_ref[...], 0.0)
    a = jnp.maximum(1.0 - jnp.sqrt(sq) * (1.0 / POINT_INFLUENCE), 0.0)
    arep = lax.dot_general(
        a, rep_ref[...], (((1,), (0,)), ((), ())),
        preferred_element_type=jnp.float32,
    )
    bd = (arep * mask_ref[...]).astype(jnp.bfloat16)
    g = g_ref[...]
    wfs = []
    for grp in range(NG):
        wfs.append(lax.dot_general(
            bd[grp * GE:(grp + 1) * GE, :], g[grp * GE:(grp + 1) * GE, :],
            (((0,), (0,)), ((), ())),
            preferred_element_type=jnp.float32,
        ))
    wf3 = jnp.concatenate(wfs, axis=0).reshape(NG, 128, D_IN)
    acc = jnp.zeros((BQ, D_OUT), jnp.float32)
    for k in range(K):
        wk = wf3[:, k * GQ:(k + 1) * GQ, :].reshape(BQ, D_IN)
        acc = acc + lax.dot_general(
            wk.astype(jnp.bfloat16), w_ref[k * D_IN:(k + 1) * D_IN, :],
            (((1,), (0,)), ((), ())),
            preferred_element_type=jnp.float32,
        )
    o_ref[...] = acc


def kernel(query_points, support_points, neighbors, x, K_points, weight):
    sp16 = jnp.pad(support_points, ((0, 0), (0, 13)))
    q16 = jnp.pad(query_points, ((0, 0), (0, 13)))
    qrep = jnp.repeat(q16, M, axis=0)
    nbf = neighbors.reshape(-1)
    xbf = x.astype(jnp.bfloat16)
    g, p16 = _sc_gather(nbf, xbf, sp16)
    kpt = jnp.pad(K_points.T, ((0, 13), (0, 1)))
    ones16 = jnp.ones((16, 16), jnp.float32)
    kpsq = jnp.pad(jnp.sum(K_points * K_points, axis=1)[None, :],
                   ((0, 0), (0, 1)), constant_values=1e6)
    rep = (jnp.arange(128)[None, :] // GQ
           == jnp.arange(16)[:, None]).astype(jnp.float32)
    mask = (jnp.arange(128)[None, :] % GQ
            == (jnp.arange(EB) // M % GQ)[:, None]).astype(jnp.float32)
    wflat = weight.reshape(K * D_IN, D_OUT).astype(jnp.bfloat16)

    out = pl.pallas_call(
        _tc_body,
        grid=(GRID,),
        in_specs=[
            pl.BlockSpec((EB, D_IN), lambda i: (i, 0)),
            pl.BlockSpec((EB, 16), lambda i: (i, 0)),
            pl.BlockSpec((EB, 16), lambda i: (i, 0)),
            pl.BlockSpec((16, 16), lambda i: (0, 0)),
            pl.BlockSpec((16, 16), lambda i: (0, 0)),
            pl.BlockSpec((1, 16), lambda i: (0, 0)),
            pl.BlockSpec((16, 128), lambda i: (0, 0)),
            pl.BlockSpec((EB, 128), lambda i: (0, 0)),
            pl.BlockSpec((K * D_IN, D_OUT), lambda i: (0, 0)),
        ],
        out_specs=pl.BlockSpec((BQ, D_OUT), lambda i: (i, 0)),
        out_shape=jax.ShapeDtypeStruct((N, D_OUT), jnp.float32),
    )(g, p16, qrep, kpt, ones16, kpsq, rep, mask, wflat)
    return out


# speedup vs baseline: 1.5811x; 1.5811x over previous
import functools

import jax
import jax.numpy as jnp
from jax import lax
from jax.experimental import pallas as pl
from jax.experimental.pallas import tpu as pltpu
from jax.experimental.pallas import tpu_sc as plsc

N = 10000
N0 = 10000
M = 32
D_IN = 128
D_OUT = 128
K = 15
POINT_INFLUENCE = 0.05

NW = 32
E = N * M
EW = E // NW
CH = 400
NCH = EW // CH

BQ = 200
EB = BQ * M
GRID = N // BQ
GQ = 8
NG = BQ // GQ
GE = GQ * M


@functools.partial(
    pl.kernel,
    out_type=[
        jax.ShapeDtypeStruct((E, D_IN), jnp.float32),
        jax.ShapeDtypeStruct((E, 128), jnp.float32),
    ],
    mesh=plsc.VectorSubcoreMesh(core_axis_name="c", subcore_axis_name="s"),
    compiler_params=pltpu.CompilerParams(use_tc_tiling_on_sc=False),
    scratch_types=[
        pltpu.VMEM((2, CH), jnp.int32),
        pltpu.VMEM((CH, D_IN), jnp.float32),
        pltpu.VMEM((CH, D_IN), jnp.float32),
        pltpu.VMEM((CH, 16), jnp.float32),
        pltpu.VMEM((CH, 16), jnp.float32),
        pltpu.SemaphoreType.DMA,
        pltpu.SemaphoreType.DMA,
        pltpu.SemaphoreType.DMA,
        pltpu.SemaphoreType.DMA,
    ],
)
def _sc_gather(nb_hbm, x_hbm, sp16_hbm, g_out, p_out,
               idx_v, gbuf0, gbuf1, pbuf0, pbuf1, sg0, sg1, sp0, sp1):
    wid = lax.axis_index("s") * 2 + lax.axis_index("c")
    base = wid * EW
    gbufs = (gbuf0, gbuf1)
    pbufs = (pbuf0, pbuf1)
    sgs = (sg0, sg1)
    sps = (sp0, sp1)

    cps = [None, None]
    for i in range(NCH):
        b = i % 2
        off = pl.multiple_of(base + i * CH, 8)
        pltpu.sync_copy(nb_hbm.at[pl.ds(off, CH)], idx_v.at[b])
        cps[b] = (
            pltpu.async_copy(x_hbm.at[idx_v.at[b]], gbufs[b], sgs[b]),
            pltpu.async_copy(sp16_hbm.at[idx_v.at[b]], pbufs[b], sps[b]),
            off,
        )
        if i > 0:
            cpg, cpp, poff = cps[1 - b]
            cpg.wait()
            cpp.wait()
            pltpu.sync_copy(gbufs[1 - b], g_out.at[pl.ds(poff, CH)])
            pltpu.sync_copy(pbufs[1 - b], p_out.at[pl.ds(poff, CH), pl.ds(0, 16)])
    b = (NCH - 1) % 2
    cpg, cpp, poff = cps[b]
    cpg.wait()
    cpp.wait()
    pltpu.sync_copy(gbufs[b], g_out.at[pl.ds(poff, CH)])
    pltpu.sync_copy(pbufs[b], p_out.at[pl.ds(poff, CH), pl.ds(0, 16)])


def _tc_body(g_ref, p_ref, qr_ref, kpt_ref, ones_ref, kpsq_ref, rep_ref,
             mask_ref, w_ref, o_ref):
    d = p_ref[:, :16] - qr_ref[...]
    cross = lax.dot_general(
        d, kpt_ref[...], (((1,), (0,)), ((), ())),
        precision=lax.Precision.HIGHEST,
        preferred_element_type=jnp.float32,
    )
    ddr = lax.dot_general(
        d * d, ones_ref[...], (((1,), (0,)), ((), ())),
        precision=lax.Precision.HIGHEST,
        preferred_element_type=jnp.float32,
    )
    sq = jnp.maximum(ddr - 2.0 * cross + kpsq_ref[...], 0.0)
    a = jnp.maximum(1.0 - jnp.sqrt(sq) * (1.0 / POINT_INFLUENCE), 0.0)
    arep = lax.dot_general(
        a, rep_ref[...], (((1,), (0,)), ((), ())),
        preferred_element_type=jnp.float32,
    )
    bd = arep * mask_ref[...]
    g = g_ref[...]
    wfs = []
    for grp in range(NG):
        wfs.append(lax.dot_general(
            bd[grp * GE:(grp + 1) * GE, :], g[grp * GE:(grp + 1) * GE, :],
            (((0,), (0,)), ((), ())),
            preferred_element_type=jnp.float32,
        ))
    wf3 = jnp.concatenate(wfs, axis=0).reshape(NG, 128, D_IN)
    acc = jnp.zeros((BQ, D_OUT), jnp.float32)
    for k in range(K):
        wk = wf3[:, k * GQ:(k + 1) * GQ, :].reshape(BQ, D_IN)
        acc = acc + lax.dot_general(
            wk, w_ref[k * D_IN:(k + 1) * D_IN, :], (((1,), (0,)), ((), ())),
            preferred_element_type=jnp.float32,
        )
    o_ref[...] = acc


def kernel(query_points, support_points, neighbors, x, K_points, weight):
    sp16 = jnp.pad(support_points, ((0, 0), (0, 13)))
    q16 = jnp.pad(query_points, ((0, 0), (0, 13)))
    qrep = jnp.repeat(q16, M, axis=0)
    nbf = neighbors.reshape(-1)
    g, p = _sc_gather(nbf, x, sp16)
    kpt = jnp.pad(K_points.T, ((0, 13), (0, 1)))
    ones16 = jnp.ones((16, 16), jnp.float32)
    kpsq = jnp.pad(jnp.sum(K_points * K_points, axis=1)[None, :],
                   ((0, 0), (0, 1)), constant_values=1e6)
    rep = (jnp.arange(128)[None, :] // GQ
           == jnp.arange(16)[:, None]).astype(jnp.float32)
    mask = (jnp.arange(128)[None, :] % GQ
            == (jnp.arange(EB) // M % GQ)[:, None]).astype(jnp.float32)
    wflat = weight.reshape(K * D_IN, D_OUT)

    out = pl.pallas_call(
        _tc_body,
        grid=(GRID,),
        in_specs=[
            pl.BlockSpec((EB, D_IN), lambda i: (i, 0)),
            pl.BlockSpec((EB, 128), lambda i: (i, 0)),
            pl.BlockSpec((EB, 16), lambda i: (i, 0)),
            pl.BlockSpec((16, 16), lambda i: (0, 0)),
            pl.BlockSpec((16, 16), lambda i: (0, 0)),
            pl.BlockSpec((1, 16), lambda i: (0, 0)),
            pl.BlockSpec((16, 128), lambda i: (0, 0)),
            pl.BlockSpec((EB, 128), lambda i: (0, 0)),
            pl.BlockSpec((K * D_IN, D_OUT), lambda i: (0, 0)),
        ],
        out_specs=pl.BlockSpec((BQ, D_OUT), lambda i: (i, 0)),
        out_shape=jax.ShapeDtypeStruct((N, D_OUT), jnp.float32),
    )(g, p, qrep, kpt, ones16, kpsq, rep, mask, wflat)
    return out


# speedup vs baseline: 2.0468x; 1.2945x over previous
import functools

import jax
import jax.numpy as jnp
from jax import lax
from jax.experimental import pallas as pl
from jax.experimental.pallas import tpu as pltpu
from jax.experimental.pallas import tpu_sc as plsc

N = 10000
N0 = 10000
M = 32
D_IN = 128
D_OUT = 128
K = 15
POINT_INFLUENCE = 0.05

NW = 32
E = N * M
EW = E // NW
CH = 400
NCH = EW // CH

BQ = 200
EB = BQ * M
GRID = N // BQ
GQ = 8
NG = BQ // GQ
GE = GQ * M


@functools.partial(
    pl.kernel,
    out_type=[
        jax.ShapeDtypeStruct((E, D_IN), jnp.float32),
        jax.ShapeDtypeStruct((E, 128), jnp.float32),
    ],
    mesh=plsc.VectorSubcoreMesh(core_axis_name="c", subcore_axis_name="s"),
    compiler_params=pltpu.CompilerParams(use_tc_tiling_on_sc=False),
    scratch_types=[
        pltpu.VMEM((2, CH), jnp.int32),
        pltpu.VMEM((CH, D_IN), jnp.float32),
        pltpu.VMEM((CH, D_IN), jnp.float32),
        pltpu.VMEM((CH, 16), jnp.float32),
        pltpu.VMEM((CH, 16), jnp.float32),
        pltpu.SemaphoreType.DMA,
        pltpu.SemaphoreType.DMA,
        pltpu.SemaphoreType.DMA,
        pltpu.SemaphoreType.DMA,
    ],
)
def _sc_gather(nb_hbm, x_hbm, sp16_hbm, g_out, p_out,
               idx_v, gbuf0, gbuf1, pbuf0, pbuf1, sg0, sg1, sp0, sp1):
    wid = lax.axis_index("s") * 2 + lax.axis_index("c")
    base = wid * EW
    gbufs = (gbuf0, gbuf1)
    pbufs = (pbuf0, pbuf1)
    sgs = (sg0, sg1)
    sps = (sp0, sp1)

    cps = [None, None]
    for i in range(NCH):
        b = i % 2
        off = pl.multiple_of(base + i * CH, 8)
        pltpu.sync_copy(nb_hbm.at[pl.ds(off, CH)], idx_v.at[b])
        cps[b] = (
            pltpu.async_copy(x_hbm.at[idx_v.at[b]], gbufs[b], sgs[b]),
            pltpu.async_copy(sp16_hbm.at[idx_v.at[b]], pbufs[b], sps[b]),
            off,
        )
        if i > 0:
            cpg, cpp, poff = cps[1 - b]
            cpg.wait()
            cpp.wait()
            pltpu.sync_copy(gbufs[1 - b], g_out.at[pl.ds(poff, CH)])
            pltpu.sync_copy(pbufs[1 - b], p_out.at[pl.ds(poff, CH), pl.ds(0, 16)])
    b = (NCH - 1) % 2
    cpg, cpp, poff = cps[b]
    cpg.wait()
    cpp.wait()
    pltpu.sync_copy(gbufs[b], g_out.at[pl.ds(poff, CH)])
    pltpu.sync_copy(pbufs[b], p_out.at[pl.ds(poff, CH), pl.ds(0, 16)])


def _tc_body(g_ref, p_ref, q_ref, v_ref, kpsq_ref, rep_ref,
             mask_ref, w_ref, o_ref):
    q = q_ref[...]
    qe = jnp.broadcast_to(q.reshape(BQ, 1, 16), (BQ, M, 16)).reshape(EB, 16)
    d = p_ref[:, :16] - qe
    u = jnp.concatenate([d * d, d], axis=1)
    sq = jnp.maximum(lax.dot_general(
        u, v_ref[...], (((1,), (0,)), ((), ())),
        precision=lax.Precision.HIGHEST,
        preferred_element_type=jnp.float32,
    ) + kpsq_ref[...], 0.0)
    a = jnp.maximum(1.0 - jnp.sqrt(sq) * (1.0 / POINT_INFLUENCE), 0.0)
    arep = lax.dot_general(
        a, rep_ref[...], (((1,), (0,)), ((), ())),
        preferred_element_type=jnp.float32,
    )
    bd = arep * mask_ref[...]
    g = g_ref[...]
    wfs = []
    for grp in range(NG):
        wfs.append(lax.dot_general(
            bd[grp * GE:(grp + 1) * GE, :], g[grp * GE:(grp + 1) * GE, :],
            (((0,), (0,)), ((), ())),
            preferred_element_type=jnp.float32,
        ))
    wf3 = jnp.concatenate(wfs, axis=0).reshape(NG, 128, D_IN)
    acc = jnp.zeros((BQ, D_OUT), jnp.float32)
    for k in range(K):
        wk = wf3[:, k * GQ:(k + 1) * GQ, :].reshape(BQ, D_IN)
        acc = acc + lax.dot_general(
            wk, w_ref[k * D_IN:(k + 1) * D_IN, :], (((1,), (0,)), ((), ())),
            preferred_element_type=jnp.float32,
        )
    o_ref[...] = acc


def kernel(query_points, support_points, neighbors, x, K_points, weight):
    sp16 = jnp.pad(support_points, ((0, 0), (0, 13)))
    q16 = jnp.pad(query_points, ((0, 0), (0, 13)))
    nbf = neighbors.reshape(-1)
    g, p = _sc_gather(nbf, x, sp16)
    kpt = jnp.pad(K_points.T, ((0, 13), (0, 1)))
    ones16 = jnp.ones((16, 16), jnp.float32)
    v = jnp.concatenate([ones16, -2.0 * kpt], axis=0)
    kpsq = jnp.pad(jnp.sum(K_points * K_points, axis=1)[None, :],
                   ((0, 0), (0, 1)), constant_values=1e6)
    rep = (jnp.arange(128)[None, :] // GQ
           == jnp.arange(16)[:, None]).astype(jnp.float32)
    mask = (jnp.arange(128)[None, :] % GQ
            == (jnp.arange(EB) // M % GQ)[:, None]).astype(jnp.float32)
    wflat = weight.reshape(K * D_IN, D_OUT)

    out = pl.pallas_call(
        _tc_body,
        grid=(GRID,),
        in_specs=[
            pl.BlockSpec((EB, D_IN), lambda i: (i, 0)),
            pl.BlockSpec((EB, 128), lambda i: (i, 0)),
            pl.BlockSpec((BQ, 16), lambda i: (i, 0)),
            pl.BlockSpec((32, 16), lambda i: (0, 0)),
            pl.BlockSpec((1, 16), lambda i: (0, 0)),
            pl.BlockSpec((16, 128), lambda i: (0, 0)),
            pl.BlockSpec((EB, 128), lambda i: (0, 0)),
            pl.BlockSpec((K * D_IN, D_OUT), lambda i: (0, 0)),
        ],
        out_specs=pl.BlockSpec((BQ, D_OUT), lambda i: (i, 0)),
        out_shape=jax.ShapeDtypeStruct((N, D_OUT), jnp.float32),
    )(g, p, q16, v, kpsq, rep, mask, wflat)
    return out


# speedup vs baseline: 2.0938x; 1.0229x over previous
import functools

import jax
import jax.numpy as jnp
from jax import lax
from jax.experimental import pallas as pl
from jax.experimental.pallas import tpu as pltpu
from jax.experimental.pallas import tpu_sc as plsc

N = 10000
N0 = 10000
M = 32
D_IN = 128
D_OUT = 128
K = 15
POINT_INFLUENCE = 0.05

NW = 32
E = N * M
EW = E // NW
CH = 400
NCH = EW // CH

BQ = 400
EB = BQ * M
GRID = N // BQ
GQ = 8
NG = BQ // GQ
GE = GQ * M


@functools.partial(
    pl.kernel,
    out_type=[
        jax.ShapeDtypeStruct((E, D_IN), jnp.float32),
        jax.ShapeDtypeStruct((E, 128), jnp.float32),
    ],
    mesh=plsc.VectorSubcoreMesh(core_axis_name="c", subcore_axis_name="s"),
    compiler_params=pltpu.CompilerParams(use_tc_tiling_on_sc=False),
    scratch_types=[
        pltpu.VMEM((2, CH), jnp.int32),
        pltpu.VMEM((CH, D_IN), jnp.float32),
        pltpu.VMEM((CH, D_IN), jnp.float32),
        pltpu.VMEM((CH, 16), jnp.float32),
        pltpu.VMEM((CH, 16), jnp.float32),
        pltpu.SemaphoreType.DMA,
        pltpu.SemaphoreType.DMA,
        pltpu.SemaphoreType.DMA,
        pltpu.SemaphoreType.DMA,
    ],
)
def _sc_gather(nb_hbm, x_hbm, sp16_hbm, g_out, p_out,
               idx_v, gbuf0, gbuf1, pbuf0, pbuf1, sg0, sg1, sp0, sp1):
    wid = lax.axis_index("s") * 2 + lax.axis_index("c")
    base = wid * EW
    gbufs = (gbuf0, gbuf1)
    pbufs = (pbuf0, pbuf1)
    sgs = (sg0, sg1)
    sps = (sp0, sp1)

    cps = [None, None]
    for i in range(NCH):
        b = i % 2
        off = pl.multiple_of(base + i * CH, 8)
        pltpu.sync_copy(nb_hbm.at[pl.ds(off, CH)], idx_v.at[b])
        cps[b] = (
            pltpu.async_copy(x_hbm.at[idx_v.at[b]], gbufs[b], sgs[b]),
            pltpu.async_copy(sp16_hbm.at[idx_v.at[b]], pbufs[b], sps[b]),
            off,
        )
        if i > 0:
            cpg, cpp, poff = cps[1 - b]
            cpg.wait()
            cpp.wait()
            pltpu.sync_copy(gbufs[1 - b], g_out.at[pl.ds(poff, CH)])
            pltpu.sync_copy(pbufs[1 - b], p_out.at[pl.ds(poff, CH), pl.ds(0, 16)])
    b = (NCH - 1) % 2
    cpg, cpp, poff = cps[b]
    cpg.wait()
    cpp.wait()
    pltpu.sync_copy(gbufs[b], g_out.at[pl.ds(poff, CH)])
    pltpu.sync_copy(pbufs[b], p_out.at[pl.ds(poff, CH), pl.ds(0, 16)])


def _tc_body(g_ref, p_ref, q_ref, v_ref, kpsq_ref, rep_ref,
             mask_ref, w_ref, o_ref):
    q = q_ref[...]
    qe = jnp.broadcast_to(q.reshape(BQ, 1, 16), (BQ, M, 16)).reshape(EB, 16)
    d = p_ref[:, :16] - qe
    u = jnp.concatenate([d * d, d], axis=1)
    sq0 = lax.dot_general(
        u, v_ref[...], (((1,), (0,)), ((), ())),
        precision=lax.Precision.HIGHEST,
        preferred_element_type=jnp.float32,
    ) + kpsq_ref[...]
    sq = 0.5 * (sq0 + jnp.abs(sq0))
    t = 1.0 - jnp.sqrt(sq) * (1.0 / POINT_INFLUENCE)
    a = 0.5 * (t + jnp.abs(t))
    arep = lax.dot_general(
        a, rep_ref[...], (((1,), (0,)), ((), ())),
        preferred_element_type=jnp.float32,
    )
    bd = arep * mask_ref[...]
    g = g_ref[...]
    wfs = []
    for grp in range(NG):
        wfs.append(lax.dot_general(
            bd[grp * GE:(grp + 1) * GE, :], g[grp * GE:(grp + 1) * GE, :],
            (((0,), (0,)), ((), ())),
            preferred_element_type=jnp.float32,
        ))
    wf3 = jnp.concatenate(wfs, axis=0).reshape(NG, 128, D_IN)
    acc = jnp.zeros((BQ, D_OUT), jnp.float32)
    for k in range(K):
        wk = wf3[:, k * GQ:(k + 1) * GQ, :].reshape(BQ, D_IN)
        acc = acc + lax.dot_general(
            wk, w_ref[k * D_IN:(k + 1) * D_IN, :], (((1,), (0,)), ((), ())),
            preferred_element_type=jnp.float32,
        )
    o_ref[...] = acc


def kernel(query_points, support_points, neighbors, x, K_points, weight):
    sp16 = jnp.pad(support_points, ((0, 0), (0, 13)))
    q16 = jnp.pad(query_points, ((0, 0), (0, 13)))
    nbf = neighbors.reshape(-1)
    g, p = _sc_gather(nbf, x, sp16)
    kpt = jnp.pad(K_points.T, ((0, 13), (0, 1)))
    ones16 = jnp.ones((16, 16), jnp.float32)
    v = jnp.concatenate([ones16, -2.0 * kpt], axis=0)
    kpsq = jnp.pad(jnp.sum(K_points * K_points, axis=1)[None, :],
                   ((0, 0), (0, 1)), constant_values=1e6)
    rep = (jnp.arange(128)[None, :] // GQ
           == jnp.arange(16)[:, None]).astype(jnp.float32)
    mask = (jnp.arange(128)[None, :] % GQ
            == (jnp.arange(EB) // M % GQ)[:, None]).astype(jnp.float32)
    wflat = weight.reshape(K * D_IN, D_OUT)

    out = pl.pallas_call(
        _tc_body,
        grid=(GRID,),
        in_specs=[
            pl.BlockSpec((EB, D_IN), lambda i: (i, 0)),
            pl.BlockSpec((EB, 128), lambda i: (i, 0)),
            pl.BlockSpec((BQ, 16), lambda i: (i, 0)),
            pl.BlockSpec((32, 16), lambda i: (0, 0)),
            pl.BlockSpec((1, 16), lambda i: (0, 0)),
            pl.BlockSpec((16, 128), lambda i: (0, 0)),
            pl.BlockSpec((EB, 128), lambda i: (0, 0)),
            pl.BlockSpec((K * D_IN, D_OUT), lambda i: (0, 0)),
        ],
        out_specs=pl.BlockSpec((BQ, D_OUT), lambda i: (i, 0)),
        out_shape=jax.ShapeDtypeStruct((N, D_OUT), jnp.float32),
    )(g, p, q16, v, kpsq, rep, mask, wflat)
    return out


# speedup vs baseline: 2.1872x; 1.0446x over previous
import functools

import jax
import jax.numpy as jnp
from jax import lax
from jax.experimental import pallas as pl
from jax.experimental.pallas import tpu as pltpu
from jax.experimental.pallas import tpu_sc as plsc

N = 10000
N0 = 10000
M = 32
D_IN = 128
D_OUT = 128
K = 15
POINT_INFLUENCE = 0.05

NW = 32
E = N * M
EH = E // 2
EW = EH // NW
CH = 200
NCH = EW // CH

BQ = 200
EB = BQ * M
GRID = (N // 2) // BQ
GQ = 8
NG = BQ // GQ
GE = GQ * M


@functools.partial(
    pl.kernel,
    out_type=[
        jax.ShapeDtypeStruct((EH, D_IN), jnp.float32),
        jax.ShapeDtypeStruct((EH, 128), jnp.float32),
    ],
    mesh=plsc.VectorSubcoreMesh(core_axis_name="c", subcore_axis_name="s"),
    compiler_params=pltpu.CompilerParams(use_tc_tiling_on_sc=False),
    scratch_types=[
        pltpu.VMEM((2, CH), jnp.int32),
        pltpu.VMEM((CH, D_IN), jnp.float32),
        pltpu.VMEM((CH, D_IN), jnp.float32),
        pltpu.VMEM((CH, 16), jnp.float32),
        pltpu.VMEM((CH, 16), jnp.float32),
        pltpu.SemaphoreType.DMA,
        pltpu.SemaphoreType.DMA,
        pltpu.SemaphoreType.DMA,
        pltpu.SemaphoreType.DMA,
    ],
)
def _sc_gather(nb_hbm, x_hbm, sp16_hbm, g_out, p_out,
               idx_v, gbuf0, gbuf1, pbuf0, pbuf1, sg0, sg1, sp0, sp1):
    wid = lax.axis_index("s") * 2 + lax.axis_index("c")
    base = wid * EW
    gbufs = (gbuf0, gbuf1)
    pbufs = (pbuf0, pbuf1)
    sgs = (sg0, sg1)
    sps = (sp0, sp1)

    cps = [None, None]
    for i in range(NCH):
        b = i % 2
        off = pl.multiple_of(base + i * CH, 8)
        pltpu.sync_copy(nb_hbm.at[pl.ds(off, CH)], idx_v.at[b])
        cps[b] = (
            pltpu.async_copy(x_hbm.at[idx_v.at[b]], gbufs[b], sgs[b]),
            pltpu.async_copy(sp16_hbm.at[idx_v.at[b]], pbufs[b], sps[b]),
            off,
        )
        if i > 0:
            cpg, cpp, poff = cps[1 - b]
            cpg.wait()
            cpp.wait()
            pltpu.sync_copy(gbufs[1 - b], g_out.at[pl.ds(poff, CH)])
            pltpu.sync_copy(pbufs[1 - b], p_out.at[pl.ds(poff, CH), pl.ds(0, 16)])
    b = (NCH - 1) % 2
    cpg, cpp, poff = cps[b]
    cpg.wait()
    cpp.wait()
    pltpu.sync_copy(gbufs[b], g_out.at[pl.ds(poff, CH)])
    pltpu.sync_copy(pbufs[b], p_out.at[pl.ds(poff, CH), pl.ds(0, 16)])


def _tc_body(g_ref, p_ref, q_ref, v_ref, kpsq_ref, rep_ref,
             mask_ref, w_ref, o_ref):
    q = q_ref[...]
    qe = jnp.broadcast_to(q.reshape(BQ, 1, 16), (BQ, M, 16)).reshape(EB, 16)
    d = p_ref[:, :16] - qe
    u = jnp.concatenate([d * d, d], axis=1)
    sq0 = lax.dot_general(
        u, v_ref[...], (((1,), (0,)), ((), ())),
        precision=lax.Precision.HIGHEST,
        preferred_element_type=jnp.float32,
    ) + kpsq_ref[...]
    sq = 0.5 * (sq0 + jnp.abs(sq0))
    t = 1.0 - jnp.sqrt(sq) * (1.0 / POINT_INFLUENCE)
    a = 0.5 * (t + jnp.abs(t))
    arep = lax.dot_general(
        a, rep_ref[...], (((1,), (0,)), ((), ())),
        preferred_element_type=jnp.float32,
    )
    bd = arep * mask_ref[...]
    g = g_ref[...]
    wfs = []
    for grp in range(NG):
        wfs.append(lax.dot_general(
            bd[grp * GE:(grp + 1) * GE, :], g[grp * GE:(grp + 1) * GE, :],
            (((0,), (0,)), ((), ())),
            preferred_element_type=jnp.float32,
        ))
    wf3 = jnp.concatenate(wfs, axis=0).reshape(NG, 128, D_IN)
    acc = jnp.zeros((BQ, D_OUT), jnp.float32)
    for k in range(K):
        wk = wf3[:, k * GQ:(k + 1) * GQ, :].reshape(BQ, D_IN)
        acc = acc + lax.dot_general(
            wk, w_ref[k * D_IN:(k + 1) * D_IN, :], (((1,), (0,)), ((), ())),
            preferred_element_type=jnp.float32,
        )
    o_ref[...] = acc


def kernel(query_points, support_points, neighbors, x, K_points, weight):
    sp16 = jnp.pad(support_points, ((0, 0), (0, 13)))
    q16 = jnp.pad(query_points, ((0, 0), (0, 13)))
    nbf = neighbors.reshape(-1)
    kpt = jnp.pad(K_points.T, ((0, 13), (0, 1)))
    ones16 = jnp.ones((16, 16), jnp.float32)
    v = jnp.concatenate([ones16, -2.0 * kpt], axis=0)
    kpsq = jnp.pad(jnp.sum(K_points * K_points, axis=1)[None, :],
                   ((0, 0), (0, 1)), constant_values=1e6)
    rep = (jnp.arange(128)[None, :] // GQ
           == jnp.arange(16)[:, None]).astype(jnp.float32)
    mask = (jnp.arange(128)[None, :] % GQ
            == (jnp.arange(EB) // M % GQ)[:, None]).astype(jnp.float32)
    wflat = weight.reshape(K * D_IN, D_OUT)

    tc_call = pl.pallas_call(
        _tc_body,
        grid=(GRID,),
        in_specs=[
            pl.BlockSpec((EB, D_IN), lambda i: (i, 0)),
            pl.BlockSpec((EB, 128), lambda i: (i, 0)),
            pl.BlockSpec((BQ, 16), lambda i: (i, 0)),
            pl.BlockSpec((32, 16), lambda i: (0, 0)),
            pl.BlockSpec((1, 16), lambda i: (0, 0)),
            pl.BlockSpec((16, 128), lambda i: (0, 0)),
            pl.BlockSpec((EB, 128), lambda i: (0, 0)),
            pl.BlockSpec((K * D_IN, D_OUT), lambda i: (0, 0)),
        ],
        out_specs=pl.BlockSpec((BQ, D_OUT), lambda i: (i, 0)),
        out_shape=jax.ShapeDtypeStruct((N // 2, D_OUT), jnp.float32),
    )

    g0, p0 = _sc_gather(nbf[:EH], x, sp16)
    g1, p1 = _sc_gather(nbf[EH:], x, sp16)
    o0 = tc_call(g0, p0, q16[:N // 2], v, kpsq, rep, mask, wflat)
    o1 = tc_call(g1, p1, q16[N // 2:], v, kpsq, rep, mask, wflat)
    return jnp.concatenate([o0, o1], axis=0)


# speedup vs baseline: 2.2731x; 1.0393x over previous
import functools

import jax
import jax.numpy as jnp
from jax import lax
from jax.experimental import pallas as pl
from jax.experimental.pallas import tpu as pltpu
from jax.experimental.pallas import tpu_sc as plsc

N = 10000
N0 = 10000
M = 32
D_IN = 128
D_OUT = 128
K = 15
POINT_INFLUENCE = 0.05

NW = 32
E = N * M
EH = E // 2
EW = EH // NW
CH = 200
NCH = EW // CH

BQ = 200
EB = BQ * M
GRID = (N // 2) // BQ
GQ = 8
NG = BQ // GQ
GE = GQ * M


@functools.partial(
    pl.kernel,
    out_type=[
        jax.ShapeDtypeStruct((EH, D_IN), jnp.float32),
        jax.ShapeDtypeStruct((EH, 128), jnp.float32),
    ],
    mesh=plsc.VectorSubcoreMesh(core_axis_name="c", subcore_axis_name="s"),
    compiler_params=pltpu.CompilerParams(use_tc_tiling_on_sc=False),
    scratch_types=[
        pltpu.VMEM((2, CH), jnp.int32),
        pltpu.VMEM((CH, D_IN), jnp.float32),
        pltpu.VMEM((CH, D_IN), jnp.float32),
        pltpu.VMEM((CH, 16), jnp.float32),
        pltpu.VMEM((CH, 16), jnp.float32),
        pltpu.SemaphoreType.DMA,
        pltpu.SemaphoreType.DMA,
        pltpu.SemaphoreType.DMA,
        pltpu.SemaphoreType.DMA,
    ],
)
def _sc_gather(nb_hbm, x_hbm, sp16_hbm, g_out, p_out,
               idx_v, gbuf0, gbuf1, pbuf0, pbuf1, sg0, sg1, sp0, sp1):
    wid = lax.axis_index("s") * 2 + lax.axis_index("c")
    base = wid * EW
    gbufs = (gbuf0, gbuf1)
    pbufs = (pbuf0, pbuf1)
    sgs = (sg0, sg1)
    sps = (sp0, sp1)

    cps = [None, None]
    for i in range(NCH):
        b = i % 2
        off = pl.multiple_of(base + i * CH, 8)
        pltpu.sync_copy(nb_hbm.at[pl.ds(off, CH)], idx_v.at[b])
        cps[b] = (
            pltpu.async_copy(x_hbm.at[idx_v.at[b]], gbufs[b], sgs[b]),
            pltpu.async_copy(sp16_hbm.at[idx_v.at[b]], pbufs[b], sps[b]),
            off,
        )
        if i > 0:
            cpg, cpp, poff = cps[1 - b]
            cpg.wait()
            cpp.wait()
            pltpu.sync_copy(gbufs[1 - b], g_out.at[pl.ds(poff, CH)])
            pltpu.sync_copy(pbufs[1 - b], p_out.at[pl.ds(poff, CH), pl.ds(0, 16)])
    b = (NCH - 1) % 2
    cpg, cpp, poff = cps[b]
    cpg.wait()
    cpp.wait()
    pltpu.sync_copy(gbufs[b], g_out.at[pl.ds(poff, CH)])
    pltpu.sync_copy(pbufs[b], p_out.at[pl.ds(poff, CH), pl.ds(0, 16)])


def _tc_body(g_ref, p_ref, q_ref, v_ref, kpsq_ref, rep_ref,
             mask_ref, w_ref, o_ref):
    q = q_ref[...]
    qe = jnp.broadcast_to(q.reshape(BQ, 1, 16), (BQ, M, 16)).reshape(EB, 16)
    d = p_ref[:, :16] - qe
    u = jnp.concatenate([d * d, d], axis=1)
    sq0 = lax.dot_general(
        u, v_ref[...], (((1,), (0,)), ((), ())),
        precision=lax.Precision.HIGHEST,
        preferred_element_type=jnp.float32,
    ) + kpsq_ref[...]
    sq = jnp.maximum(sq0, 0.0)
    a = jnp.maximum(1.0 - jnp.sqrt(sq) * (1.0 / POINT_INFLUENCE), 0.0)
    arep = lax.dot_general(
        a, rep_ref[...], (((1,), (0,)), ((), ())),
        preferred_element_type=jnp.float32,
    )
    bd = arep * mask_ref[...]
    g = g_ref[...]
    wfs = []
    for grp in range(NG):
        wfs.append(lax.dot_general(
            bd[grp * GE:(grp + 1) * GE, :], g[grp * GE:(grp + 1) * GE, :],
            (((0,), (0,)), ((), ())),
            preferred_element_type=jnp.float32,
        ))
    wf3 = jnp.concatenate(wfs, axis=0).reshape(NG, 128, D_IN)
    acc = jnp.zeros((BQ, D_OUT), jnp.float32)
    for k in range(K):
        wk = wf3[:, k * GQ:(k + 1) * GQ, :].reshape(BQ, D_IN)
        acc = acc + lax.dot_general(
            wk, w_ref[k * D_IN:(k + 1) * D_IN, :], (((1,), (0,)), ((), ())),
            preferred_element_type=jnp.float32,
        )
    o_ref[...] = acc


def kernel(query_points, support_points, neighbors, x, K_points, weight):
    sp16 = jnp.pad(support_points, ((0, 0), (0, 13)))
    q16 = jnp.pad(query_points, ((0, 0), (0, 13)))
    nbf = neighbors.reshape(-1)
    kpt = jnp.pad(K_points.T, ((0, 13), (0, 1)))
    ones16 = jnp.ones((16, 16), jnp.float32)
    v = jnp.concatenate([ones16, -2.0 * kpt], axis=0)
    kpsq = jnp.pad(jnp.sum(K_points * K_points, axis=1)[None, :],
                   ((0, 0), (0, 1)), constant_values=1e6)
    rep = (jnp.arange(128)[None, :] // GQ
           == jnp.arange(16)[:, None]).astype(jnp.float32)
    mask = (jnp.arange(128)[None, :] % GQ
            == (jnp.arange(EB) // M % GQ)[:, None]).astype(jnp.float32)
    wflat = weight.reshape(K * D_IN, D_OUT)

    tc_call = pl.pallas_call(
        _tc_body,
        grid=(GRID,),
        in_specs=[
            pl.BlockSpec((EB, D_IN), lambda i: (i, 0)),
            pl.BlockSpec((EB, 128), lambda i: (i, 0)),
            pl.BlockSpec((BQ, 16), lambda i: (i, 0)),
            pl.BlockSpec((32, 16), lambda i: (0, 0)),
            pl.BlockSpec((1, 16), lambda i: (0, 0)),
            pl.BlockSpec((16, 128), lambda i: (0, 0)),
            pl.BlockSpec((EB, 128), lambda i: (0, 0)),
            pl.BlockSpec((K * D_IN, D_OUT), lambda i: (0, 0)),
        ],
        out_specs=pl.BlockSpec((BQ, D_OUT), lambda i: (i, 0)),
        out_shape=jax.ShapeDtypeStruct((N // 2, D_OUT), jnp.float32),
    )

    g0, p0 = _sc_gather(nbf[:EH], x, sp16)
    g1, p1 = _sc_gather(nbf[EH:], x, sp16)
    o0 = tc_call(g0, p0, q16[:N // 2], v, kpsq, rep, mask, wflat)
    o1 = tc_call(g1, p1, q16[N // 2:], v, kpsq, rep, mask, wflat)
    return jnp.concatenate([o0, o1], axis=0)
